# baseline (speedup 1.0000x reference)
import jax
import jax.numpy as jnp
from jax.experimental import pallas as pl
from jax.experimental.pallas import tpu as pltpu

_F = 4
_TS = 262144


def _round_up(v, m):
    return ((v + m - 1) // m) * m


def _split_hi_lo(a):
    bits = jax.lax.bitcast_convert_type(a, jnp.uint32)
    hi = jax.lax.bitcast_convert_type(
        bits & jnp.uint32(0xFFFF0000), jnp.float32)
    return hi, a - hi


def _body(x_ref, a_ref, o_ref):
    xb = x_ref[...]
    x8 = jnp.concatenate([xb[:, :_TS], xb[:, _TS:]], axis=0)
    hi, lo = _split_hi_lo(x8)
    hi = hi.astype(jnp.bfloat16)
    lo = lo.astype(jnp.bfloat16)
    ones = jnp.ones_like(hi[0:4])
    rhs = jnp.concatenate([hi, lo, hi, ones], axis=0)
    y = jnp.dot(a_ref[...], rhs, preferred_element_type=jnp.float32)
    g = jnp.exp(-50.0 * (y * y))
    o_ref[:, :_TS] = g[0:4]
    o_ref[:, _TS:] = g[4:8]


def kernel(x, w, b):
    B, f_in = x.shape
    f_out = w.shape[1]
    assert f_in == _F and f_out == _F

    group = 2 * _TS
    pB = _round_up(B, group)
    xt = x.T
    if pB != B:
        xt = jnp.pad(xt, ((0, 0), (0, pB - B)))

    wh, wl = _split_hi_lo(w)
    bh, bl = _split_hi_lo(b)
    eye2 = jnp.eye(2, dtype=jnp.float32)
    bh2 = jnp.tile(bh.reshape(_F, 1), (2, 1))
    bl2 = jnp.tile(bl.reshape(_F, 1), (2, 1))
    a28 = jnp.concatenate(
        [jnp.kron(eye2, wh.T), jnp.kron(eye2, wh.T), jnp.kron(eye2, wl.T),
         bh2, bl2, jnp.zeros((8, 2), jnp.float32)], axis=1)
    a28 = a28.astype(jnp.bfloat16)

    grid = (pB // group,)

    out_t = pl.pallas_call(
        _body,
        out_shape=jax.ShapeDtypeStruct((_F, pB), jnp.float32),
        grid=grid,
        in_specs=[
            pl.BlockSpec((_F, group), lambda i: (0, i)),
            pl.BlockSpec((8, 28), lambda i: (0, 0)),
        ],
        out_specs=pl.BlockSpec((_F, group), lambda i: (0, i)),
        compiler_params=pltpu.CompilerParams(
            dimension_semantics=("arbitrary",),
            vmem_limit_bytes=56 * 1024 * 1024,
        ),
        cost_estimate=pl.CostEstimate(
            flops=2 * pB * 28 * 8,
            transcendentals=pB * _F,
            bytes_accessed=2 * pB * _F * 4,
        ),
    )(xt, a28)

    return out_t[:, :B].T
